# Initial kernel scaffold; baseline (speedup 1.0000x reference)
import jax
import jax.numpy as jnp
from jax.experimental import pallas as pl


def kernel(x, city, delta_t_info, delta_dis_info, delta_rg_info, delta_entropy_info, city_embeddings, router_w, router_b, fc_w, fc_b, proj_w, proj_b):
    raise NotImplementedError("write your pallas kernel here")


# speedup vs baseline: 1.1727x; 1.1727x over previous
import functools

import jax
import jax.numpy as jnp
from jax.experimental import pallas as pl
from jax.experimental.pallas import tpu as pltpu

B, T, N_EMBD = 4, 2048, 768
NUM_EXPERTS, TOP_K = 8, 2
CITY_DIM = 32
D_FF = 4 * N_EMBD
N_TOK = B * T

_SQRT_2_OVER_PI = 0.7978845608028654


def _gelu_tanh(x):
    return 0.5 * x * (1.0 + jnp.tanh(_SQRT_2_OVER_PI * (x + 0.044715 * x * x * x)))


def _router_body(x_ref, d1_ref, d2_ref, d3_ref, d4_ref,
                 wx_ref, w1_ref, w2_ref, w3_ref, w4_ref, bias_ref,
                 gate1_ref, gating_ref):
    logits = jnp.dot(x_ref[...], wx_ref[...], preferred_element_type=jnp.float32)
    logits += jnp.dot(d1_ref[...], w1_ref[...], preferred_element_type=jnp.float32)
    logits += jnp.dot(d2_ref[...], w2_ref[...], preferred_element_type=jnp.float32)
    logits += jnp.dot(d3_ref[...], w3_ref[...], preferred_element_type=jnp.float32)
    logits += jnp.dot(d4_ref[...], w4_ref[...], preferred_element_type=jnp.float32)
    logits += bias_ref[...]

    m = jnp.max(logits, axis=-1, keepdims=True)
    e = jnp.exp(logits - m)
    gate1_ref[...] = e / jnp.sum(e, axis=-1, keepdims=True)

    iota = jax.lax.broadcasted_iota(jnp.int32, logits.shape, 1)
    m1 = jnp.max(logits, axis=-1, keepdims=True)
    is1 = logits == m1
    i1 = jnp.min(jnp.where(is1, iota, NUM_EXPERTS), axis=-1, keepdims=True)
    masked = jnp.where(iota == i1, -jnp.inf, logits)
    m2 = jnp.max(masked, axis=-1, keepdims=True)
    i2 = jnp.min(jnp.where(masked == m2, iota, NUM_EXPERTS), axis=-1, keepdims=True)
    w_top1 = 1.0 / (1.0 + jnp.exp(m2 - m1))
    w_top2 = 1.0 - w_top1
    gating_ref[...] = (jnp.where(iota == i1, w_top1, 0.0)
                       + jnp.where(iota == i2, w_top2, 0.0))


def _run_router(x2d, d1, d2, d3, d4, router_w, bias_full):
    blk = 1024
    grid = (N_TOK // blk,)
    wx = router_w[:N_EMBD]
    o = N_EMBD + CITY_DIM
    w1 = router_w[o:o + 192]
    w2 = router_w[o + 192:o + 384]
    w3 = router_w[o + 384:o + 480]
    w4 = router_w[o + 480:o + 576]

    def tok_block(i):
        return (i, 0)

    def full(i):
        return (0, 0)

    return pl.pallas_call(
        _router_body,
        grid=grid,
        in_specs=[
            pl.BlockSpec((blk, N_EMBD), tok_block),
            pl.BlockSpec((blk, 192), tok_block),
            pl.BlockSpec((blk, 192), tok_block),
            pl.BlockSpec((blk, 96), tok_block),
            pl.BlockSpec((blk, 96), tok_block),
            pl.BlockSpec((N_EMBD, NUM_EXPERTS), full),
            pl.BlockSpec((192, NUM_EXPERTS), full),
            pl.BlockSpec((192, NUM_EXPERTS), full),
            pl.BlockSpec((96, NUM_EXPERTS), full),
            pl.BlockSpec((96, NUM_EXPERTS), full),
            pl.BlockSpec((1, NUM_EXPERTS), full),
        ],
        out_specs=[
            pl.BlockSpec((blk, NUM_EXPERTS), tok_block),
            pl.BlockSpec((blk, NUM_EXPERTS), tok_block),
        ],
        out_shape=[
            jax.ShapeDtypeStruct((N_TOK, NUM_EXPERTS), jnp.float32),
            jax.ShapeDtypeStruct((N_TOK, NUM_EXPERTS), jnp.float32),
        ],
    )(x2d, d1, d2, d3, d4, wx, w1, w2, w3, w4, bias_full)


_N_FF_CHUNK = 2
_FF_CHUNK = D_FF // _N_FF_CHUNK


def _ffn_body(x_ref, g_ref, fcw_ref, fcb_ref, pjw_ref, pjb_ref, out_ref,
              acc_ref):
    e = pl.program_id(0)
    f = pl.program_id(1)
    i = pl.program_id(2)
    blk = x_ref.shape[0]
    sel = (jax.lax.broadcasted_iota(jnp.int32, (1, NUM_EXPERTS), 1)
           == e).astype(jnp.float32)
    ge = jnp.sum(g_ref[...] * sel, axis=-1, keepdims=True)
    hid = jnp.dot(x_ref[...], fcw_ref[0], preferred_element_type=jnp.float32)
    hid = _gelu_tanh(hid + fcb_ref[0])
    out = jnp.dot(hid, pjw_ref[0], preferred_element_type=jnp.float32)
    out = ge * jnp.where(f == 0, out + pjb_ref[0], out)
    sl = pl.ds(i * blk, blk)
    first = jnp.logical_and(e == 0, f == 0)
    @pl.when(first)
    def _():
        acc_ref[sl, :] = out
    @pl.when(jnp.logical_not(first))
    def _():
        acc_ref[sl, :] += out
    @pl.when(jnp.logical_and(e == NUM_EXPERTS - 1, f == _N_FF_CHUNK - 1))
    def _():
        out_ref[...] = acc_ref[sl, :]


def _run_dense_ffn(x2d, gating, fc_w, fc_b, proj_w, proj_b):
    blk = 512
    nb = N_TOK // blk
    grid = (NUM_EXPERTS, _N_FF_CHUNK, nb)
    return pl.pallas_call(
        _ffn_body,
        grid=grid,
        in_specs=[
            pl.BlockSpec((blk, N_EMBD), lambda e, f, i: (i, 0)),
            pl.BlockSpec((blk, NUM_EXPERTS), lambda e, f, i: (i, 0)),
            pl.BlockSpec((1, N_EMBD, _FF_CHUNK), lambda e, f, i: (e, 0, f)),
            pl.BlockSpec((1, 1, _FF_CHUNK), lambda e, f, i: (e, 0, f)),
            pl.BlockSpec((1, _FF_CHUNK, N_EMBD), lambda e, f, i: (e, f, 0)),
            pl.BlockSpec((1, 1, N_EMBD), lambda e, f, i: (e, 0, 0)),
        ],
        out_specs=pl.BlockSpec((blk, N_EMBD), lambda e, f, i: (i, 0)),
        out_shape=jax.ShapeDtypeStruct((N_TOK, N_EMBD), jnp.float32),
        scratch_shapes=[pltpu.VMEM((N_TOK, N_EMBD), jnp.float32)],
    )(x2d, gating, fc_w, fc_b.reshape(NUM_EXPERTS, 1, D_FF),
      proj_w, proj_b.reshape(NUM_EXPERTS, 1, N_EMBD))


def kernel(x, city, delta_t_info, delta_dis_info, delta_rg_info,
           delta_entropy_info, city_embeddings, router_w, router_b,
           fc_w, fc_b, proj_w, proj_b):
    b, t, d = x.shape
    n = b * t
    x2d = x.reshape(n, d)
    d1 = delta_t_info.reshape(n, -1)
    d2 = delta_dis_info.reshape(n, -1)
    d3 = delta_rg_info.reshape(n, -1)
    d4 = delta_entropy_info.reshape(n, -1)

    ce = city_embeddings[city]
    w_ce = jax.lax.dynamic_slice_in_dim(router_w, N_EMBD, CITY_DIM, 0)
    bias_full = (jnp.dot(ce, w_ce) + router_b).reshape(1, NUM_EXPERTS)

    gate1, gating = _run_router(x2d, d1, d2, d3, d4, router_w, bias_full)
    final = _run_dense_ffn(x2d, gating, fc_w, fc_b, proj_w, proj_b)
    return final.reshape(b, t, d), gate1.reshape(b, t, NUM_EXPERTS)


# speedup vs baseline: 2.0490x; 1.7473x over previous
import functools

import jax
import jax.numpy as jnp
from jax import lax
from jax.experimental import pallas as pl
from jax.experimental.pallas import tpu as pltpu
from jax.experimental.pallas import tpu_sc as plsc

B, T, N_EMBD = 4, 2048, 768
NUM_EXPERTS, TOP_K = 8, 2
CITY_DIM = 32
D_FF = 4 * N_EMBD
N_TOK = B * T
N_SLOT = N_TOK * TOP_K

BLK = 256
NBLK = 72
PADN = NBLK * BLK

_NC, _NS = 2, 16
_NW = _NC * _NS
_SLOT_PER_W = N_SLOT // _NW
_TOK_PER_W = _SLOT_PER_W

_SQRT_2_OVER_PI = 0.7978845608028654


def _gelu_tanh(x):
    return 0.5 * x * (1.0 + jnp.tanh(_SQRT_2_OVER_PI * (x + 0.044715 * x * x * x)))


def _router_body(x_ref, d1_ref, d2_ref, d3_ref, d4_ref,
                 wx_ref, w1_ref, w2_ref, w3_ref, w4_ref, bias_ref,
                 gate1_ref, i1_ref, i2_ref, w1o_ref, w2o_ref):
    logits = jnp.dot(x_ref[...], wx_ref[...], preferred_element_type=jnp.float32)
    logits += jnp.dot(d1_ref[...], w1_ref[...], preferred_element_type=jnp.float32)
    logits += jnp.dot(d2_ref[...], w2_ref[...], preferred_element_type=jnp.float32)
    logits += jnp.dot(d3_ref[...], w3_ref[...], preferred_element_type=jnp.float32)
    logits += jnp.dot(d4_ref[...], w4_ref[...], preferred_element_type=jnp.float32)
    logits += bias_ref[...]

    m1 = jnp.max(logits, axis=-1, keepdims=True)
    e = jnp.exp(logits - m1)
    gate1_ref[...] = e / jnp.sum(e, axis=-1, keepdims=True)

    iota = jax.lax.broadcasted_iota(jnp.int32, logits.shape, 1)
    i1 = jnp.min(jnp.where(logits == m1, iota, NUM_EXPERTS), axis=-1,
                 keepdims=True)
    masked = jnp.where(iota == i1, -jnp.inf, logits)
    m2 = jnp.max(masked, axis=-1, keepdims=True)
    i2 = jnp.min(jnp.where(masked == m2, iota, NUM_EXPERTS), axis=-1,
                 keepdims=True)
    w_top1 = 1.0 / (1.0 + jnp.exp(m2 - m1))
    i1_ref[...] = i1
    i2_ref[...] = i2
    w1o_ref[...] = w_top1
    w2o_ref[...] = 1.0 - w_top1


def _run_router(x2d, d1, d2, d3, d4, router_w, bias_full):
    blk = 1024
    grid = (N_TOK // blk,)
    wx = router_w[:N_EMBD]
    o = N_EMBD + CITY_DIM
    w1 = router_w[o:o + 192]
    w2 = router_w[o + 192:o + 384]
    w3 = router_w[o + 384:o + 480]
    w4 = router_w[o + 480:o + 576]

    def tok_block(i):
        return (i, 0)

    def full(i):
        return (0, 0)

    col = pl.BlockSpec((blk, 1), tok_block)
    return pl.pallas_call(
        _router_body,
        grid=grid,
        in_specs=[
            pl.BlockSpec((blk, N_EMBD), tok_block),
            pl.BlockSpec((blk, 192), tok_block),
            pl.BlockSpec((blk, 192), tok_block),
            pl.BlockSpec((blk, 96), tok_block),
            pl.BlockSpec((blk, 96), tok_block),
            pl.BlockSpec((N_EMBD, NUM_EXPERTS), full),
            pl.BlockSpec((192, NUM_EXPERTS), full),
            pl.BlockSpec((192, NUM_EXPERTS), full),
            pl.BlockSpec((96, NUM_EXPERTS), full),
            pl.BlockSpec((96, NUM_EXPERTS), full),
            pl.BlockSpec((1, NUM_EXPERTS), full),
        ],
        out_specs=[
            pl.BlockSpec((blk, NUM_EXPERTS), tok_block),
            col, col, col, col,
        ],
        out_shape=[
            jax.ShapeDtypeStruct((N_TOK, NUM_EXPERTS), jnp.float32),
            jax.ShapeDtypeStruct((N_TOK, 1), jnp.int32),
            jax.ShapeDtypeStruct((N_TOK, 1), jnp.int32),
            jax.ShapeDtypeStruct((N_TOK, 1), jnp.float32),
            jax.ShapeDtypeStruct((N_TOK, 1), jnp.float32),
        ],
    )(x2d, d1, d2, d3, d4, wx, w1, w2, w3, w4, bias_full)


_GRP = 16
_CHUNK = 128
_N_CHUNK = _SLOT_PER_W // _CHUNK
_POS_GRPS = _SLOT_PER_W // _GRP


def _sc_count_body(ex_hbm, cnt_hbm, ex_v, cnt_stage):
    c = lax.axis_index("c")
    s = lax.axis_index("s")
    w = 2 * s + c
    lane = lax.iota(jnp.int32, 16)
    pltpu.sync_copy(ex_hbm.at[pl.ds(512 * w, 512)], ex_v.at[pl.ds(0, 512)])
    zero16 = jnp.zeros((16,), jnp.int32)

    def body(g, a):
        v = ex_v[pl.ds(g * 16, 16)]
        for e in range(NUM_EXPERTS):
            p = jnp.sum((v == e).astype(jnp.int32))
            a = a + jnp.where(lane == e, p, 0)
        return a

    cnt = lax.fori_loop(0, 512 // 16, body, zero16)
    cnt_stage[0, :] = cnt
    pltpu.sync_copy(cnt_stage, cnt_hbm.at[pl.ds(w, 1)])


def _run_sc_count(ex_flat):
    mesh = plsc.VectorSubcoreMesh(core_axis_name="c", subcore_axis_name="s")
    f = pl.kernel(
        _sc_count_body,
        out_type=jax.ShapeDtypeStruct((_NW, 16), jnp.int32),
        mesh=mesh,
        scratch_types=[
            pltpu.VMEM((1024,), jnp.int32),
            pltpu.VMEM((1, 16), jnp.int32),
        ],
        compiler_params=pltpu.CompilerParams(needs_layout_passes=False),
    )
    return f(ex_flat)


def _sc_route_body(ex_hbm, gw_hbm, x_hbm, cnt_hbm,
                   pos_hbm, gwp_hbm, be_hbm, px_hbm,
                   ex_v, all_v, pos_v, gw_v, be_v, xbuf, sem):
    c = lax.axis_index("c")
    s = lax.axis_index("s")
    w = 2 * s + c
    lane = lax.iota(jnp.int32, 16)
    zero16 = jnp.zeros((16,), jnp.int32)

    pltpu.sync_copy(cnt_hbm, all_v)
    tot = zero16
    pre = zero16
    for r in range(_NW):
        row = all_v[r, :]
        tot = tot + row
        pre = pre + row * (r < w).astype(jnp.int32)
    padded = ((tot + (BLK - 1)) // BLK) * BLK
    base_excl = plsc.cumsum(padded) - padded
    start = base_excl + pre

    pltpu.sync_copy(ex_hbm.at[pl.ds(512 * w, 512)], ex_v.at[pl.ds(0, 512)])

    def pos_grp(g, start_vec):
        v = ex_v[pl.ds(g * 16, 16)]
        posv = jnp.zeros((16,), jnp.int32)
        upd = start_vec
        for e in range(NUM_EXPERTS):
            mi = (v == e).astype(jnp.int32)
            csum = plsc.cumsum(mi)
            start_e = jnp.sum(jnp.where(lane == e, start_vec, 0))
            posv = posv + mi * (start_e + csum - 1)
            cnt_e = jnp.sum(mi)
            upd = upd + jnp.where(lane == e, cnt_e, 0)
        pos_v[g // 8, pl.ds((g % 8) * 16, 16)] = posv
        return upd

    lax.fori_loop(0, _POS_GRPS, pos_grp, start)

    for r in range(4):
        pltpu.sync_copy(pos_v.at[r], pos_hbm.at[pl.ds(512 * w + 128 * r, 128)])

    for r in range(4):
        pltpu.sync_copy(gw_hbm.at[pl.ds(512 * w + 128 * r, 128)], gw_v.at[r])
    for r in range(4):
        pltpu.async_copy(gw_v.at[r], gwp_hbm.at[pos_v.at[r]], sem).wait()

    @pl.when(w == 0)
    def _():
        base_blk = base_excl // BLK
        sb = [jnp.sum(jnp.where(lane == e, base_blk, 0))
              for e in range(NUM_EXPERTS)]
        for j in range(128 // 16):
            blkid = lane + 16 * j
            bev = jnp.zeros((16,), jnp.int32)
            for e in range(NUM_EXPERTS):
                bev = bev + (blkid >= sb[e]).astype(jnp.int32)
            be_v[pl.ds(16 * j, 16)] = bev - 1
        pltpu.sync_copy(be_v, be_hbm)

    tok_base = 512 * (w % 16)
    for cchunk in range(_N_CHUNK):
        pltpu.sync_copy(x_hbm.at[pl.ds(tok_base + 128 * cchunk, 128)], xbuf)
        pltpu.async_copy(xbuf, px_hbm.at[pos_v.at[cchunk]], sem).wait()


def _run_sc_route(ex_flat, gw_flat, x2d, cnt):
    mesh = plsc.VectorSubcoreMesh(core_axis_name="c", subcore_axis_name="s")
    f = pl.kernel(
        _sc_route_body,
        out_type=[
            jax.ShapeDtypeStruct((N_SLOT,), jnp.int32),
            jax.ShapeDtypeStruct((PADN,), jnp.float32),
            jax.ShapeDtypeStruct((128,), jnp.int32),
            jax.ShapeDtypeStruct((PADN, N_EMBD), jnp.float32),
        ],
        mesh=mesh,
        scratch_types=[
            pltpu.VMEM((1024,), jnp.int32),
            pltpu.VMEM((_NW, 16), jnp.int32),
            pltpu.VMEM((4, 128), jnp.int32),
            pltpu.VMEM((4, 128), jnp.float32),
            pltpu.VMEM((128,), jnp.int32),
            pltpu.VMEM((128, N_EMBD), jnp.float32),
            pltpu.SemaphoreType.DMA,
        ],
        compiler_params=pltpu.CompilerParams(needs_layout_passes=False),
    )
    return f(ex_flat, gw_flat, x2d, cnt)


def _gffn_body(be_ref, x_ref, gw_ref, fcw_ref, fcb_ref, pjw_ref, pjb_ref,
               out_ref):
    hid = jnp.dot(x_ref[...], fcw_ref[0], preferred_element_type=jnp.float32)
    hid = _gelu_tanh(hid + fcb_ref[0])
    out = jnp.dot(hid, pjw_ref[0], preferred_element_type=jnp.float32)
    out_ref[...] = (out + pjb_ref[0]) * gw_ref[...]


def _run_grouped_ffn(be, perm_x, gw_perm, fc_w, fc_b, proj_w, proj_b):
    grid_spec = pltpu.PrefetchScalarGridSpec(
        num_scalar_prefetch=1,
        grid=(NBLK,),
        in_specs=[
            pl.BlockSpec((BLK, N_EMBD), lambda g, be: (g, 0)),
            pl.BlockSpec((BLK, 1), lambda g, be: (g, 0)),
            pl.BlockSpec((1, N_EMBD, D_FF), lambda g, be: (be[g], 0, 0)),
            pl.BlockSpec((1, 1, D_FF), lambda g, be: (be[g], 0, 0)),
            pl.BlockSpec((1, D_FF, N_EMBD), lambda g, be: (be[g], 0, 0)),
            pl.BlockSpec((1, 1, N_EMBD), lambda g, be: (be[g], 0, 0)),
        ],
        out_specs=pl.BlockSpec((BLK, N_EMBD), lambda g, be: (g, 0)),
    )
    return pl.pallas_call(
        _gffn_body,
        grid_spec=grid_spec,
        out_shape=jax.ShapeDtypeStruct((PADN, N_EMBD), jnp.float32),
    )(be, perm_x, gw_perm.reshape(PADN, 1), fc_w,
      fc_b.reshape(NUM_EXPERTS, 1, D_FF), proj_w,
      proj_b.reshape(NUM_EXPERTS, 1, N_EMBD))


_CTOK = 32


def _sc_combine_body(yw_hbm, pos_hbm, out_hbm,
                     idx0_v, idx1_v, buf0, buf1, sem):
    c = lax.axis_index("c")
    s = lax.axis_index("s")
    w = 2 * s + c
    tok_base = _TOK_PER_W // 2 * w

    for ch in range(256 // _CTOK):
        t0 = tok_base + _CTOK * ch
        pltpu.sync_copy(pos_hbm.at[pl.ds(t0, _CTOK)], idx0_v)
        pltpu.sync_copy(pos_hbm.at[pl.ds(N_TOK + t0, _CTOK)], idx1_v)
        cp0 = pltpu.async_copy(yw_hbm.at[idx0_v], buf0, sem)
        cp1 = pltpu.async_copy(yw_hbm.at[idx1_v], buf1, sem)
        cp0.wait()
        cp1.wait()

        def add_row(j, _):
            r = j // (N_EMBD // 16)
            col = (j % (N_EMBD // 16)) * 16
            buf0[r, pl.ds(col, 16)] = (buf0[r, pl.ds(col, 16)]
                                       + buf1[r, pl.ds(col, 16)])
            return 0

        lax.fori_loop(0, _CTOK * (N_EMBD // 16), add_row, 0)
        pltpu.sync_copy(buf0, out_hbm.at[pl.ds(t0, _CTOK)])


def _run_sc_combine(yw, pos):
    mesh = plsc.VectorSubcoreMesh(core_axis_name="c", subcore_axis_name="s")
    f = pl.kernel(
        _sc_combine_body,
        out_type=jax.ShapeDtypeStruct((N_TOK, N_EMBD), jnp.float32),
        mesh=mesh,
        scratch_types=[
            pltpu.VMEM((_CTOK,), jnp.int32),
            pltpu.VMEM((_CTOK,), jnp.int32),
            pltpu.VMEM((_CTOK, N_EMBD), jnp.float32),
            pltpu.VMEM((_CTOK, N_EMBD), jnp.float32),
            pltpu.SemaphoreType.DMA,
        ],
        compiler_params=pltpu.CompilerParams(needs_layout_passes=False),
    )
    return f(yw, pos)


def kernel(x, city, delta_t_info, delta_dis_info, delta_rg_info,
           delta_entropy_info, city_embeddings, router_w, router_b,
           fc_w, fc_b, proj_w, proj_b):
    b, t, d = x.shape
    n = b * t
    x2d = x.reshape(n, d)
    d1 = delta_t_info.reshape(n, -1)
    d2 = delta_dis_info.reshape(n, -1)
    d3 = delta_rg_info.reshape(n, -1)
    d4 = delta_entropy_info.reshape(n, -1)

    ce = city_embeddings[city]
    w_ce = jax.lax.dynamic_slice_in_dim(router_w, N_EMBD, CITY_DIM, 0)
    bias_full = (jnp.dot(ce, w_ce) + router_b).reshape(1, NUM_EXPERTS)

    gate1, i1, i2, w1, w2 = _run_router(x2d, d1, d2, d3, d4, router_w,
                                        bias_full)
    ex_flat = jnp.concatenate([i1, i2], axis=0).reshape(-1)
    gw_flat = jnp.concatenate([w1, w2], axis=0).reshape(-1)

    cnt = _run_sc_count(ex_flat)
    pos, gw_perm, be_pad, perm_x = _run_sc_route(ex_flat, gw_flat, x2d, cnt)
    yw = _run_grouped_ffn(be_pad[:NBLK], perm_x, gw_perm,
                          fc_w, fc_b, proj_w, proj_b)
    out2d = _run_sc_combine(yw, pos)
    return out2d.reshape(b, t, d), gate1.reshape(b, t, NUM_EXPERTS)


# speedup vs baseline: 2.0952x; 1.0225x over previous
import functools

import jax
import jax.numpy as jnp
from jax import lax
from jax.experimental import pallas as pl
from jax.experimental.pallas import tpu as pltpu
from jax.experimental.pallas import tpu_sc as plsc

B, T, N_EMBD = 4, 2048, 768
NUM_EXPERTS, TOP_K = 8, 2
CITY_DIM = 32
D_FF = 4 * N_EMBD
N_TOK = B * T
N_SLOT = N_TOK * TOP_K

BLK = 256
NBLK = 72
PADN = NBLK * BLK

_NC, _NS = 2, 16
_NW = _NC * _NS
_SLOT_PER_W = N_SLOT // _NW
_TOK_PER_W = _SLOT_PER_W

_SQRT_2_OVER_PI = 0.7978845608028654


def _gelu_tanh(x):
    return 0.5 * x * (1.0 + jnp.tanh(_SQRT_2_OVER_PI * (x + 0.044715 * x * x * x)))


def _router_body(x_ref, d1_ref, d2_ref, d3_ref, d4_ref,
                 wx_ref, w1_ref, w2_ref, w3_ref, w4_ref, bias_ref,
                 gate1_ref, i1_ref, i2_ref, w1o_ref, w2o_ref):
    logits = jnp.dot(x_ref[...], wx_ref[...], preferred_element_type=jnp.float32)
    logits += jnp.dot(d1_ref[...], w1_ref[...], preferred_element_type=jnp.float32)
    logits += jnp.dot(d2_ref[...], w2_ref[...], preferred_element_type=jnp.float32)
    logits += jnp.dot(d3_ref[...], w3_ref[...], preferred_element_type=jnp.float32)
    logits += jnp.dot(d4_ref[...], w4_ref[...], preferred_element_type=jnp.float32)
    logits += bias_ref[...]

    m1 = jnp.max(logits, axis=-1, keepdims=True)
    e = jnp.exp(logits - m1)
    gate1_ref[...] = e / jnp.sum(e, axis=-1, keepdims=True)

    iota = jax.lax.broadcasted_iota(jnp.int32, logits.shape, 1)
    i1 = jnp.min(jnp.where(logits == m1, iota, NUM_EXPERTS), axis=-1,
                 keepdims=True)
    masked = jnp.where(iota == i1, -jnp.inf, logits)
    m2 = jnp.max(masked, axis=-1, keepdims=True)
    i2 = jnp.min(jnp.where(masked == m2, iota, NUM_EXPERTS), axis=-1,
                 keepdims=True)
    w_top1 = 1.0 / (1.0 + jnp.exp(m2 - m1))
    i1_ref[...] = i1
    i2_ref[...] = i2
    w1o_ref[...] = w_top1
    w2o_ref[...] = 1.0 - w_top1


def _run_router(x2d, d1, d2, d3, d4, router_w, bias_full):
    blk = 1024
    grid = (N_TOK // blk,)
    wx = router_w[:N_EMBD]
    o = N_EMBD + CITY_DIM
    w1 = router_w[o:o + 192]
    w2 = router_w[o + 192:o + 384]
    w3 = router_w[o + 384:o + 480]
    w4 = router_w[o + 480:o + 576]

    def tok_block(i):
        return (i, 0)

    def full(i):
        return (0, 0)

    col = pl.BlockSpec((blk, 1), tok_block)
    return pl.pallas_call(
        _router_body,
        grid=grid,
        in_specs=[
            pl.BlockSpec((blk, N_EMBD), tok_block),
            pl.BlockSpec((blk, 192), tok_block),
            pl.BlockSpec((blk, 192), tok_block),
            pl.BlockSpec((blk, 96), tok_block),
            pl.BlockSpec((blk, 96), tok_block),
            pl.BlockSpec((N_EMBD, NUM_EXPERTS), full),
            pl.BlockSpec((192, NUM_EXPERTS), full),
            pl.BlockSpec((192, NUM_EXPERTS), full),
            pl.BlockSpec((96, NUM_EXPERTS), full),
            pl.BlockSpec((96, NUM_EXPERTS), full),
            pl.BlockSpec((1, NUM_EXPERTS), full),
        ],
        out_specs=[
            pl.BlockSpec((blk, NUM_EXPERTS), tok_block),
            col, col, col, col,
        ],
        out_shape=[
            jax.ShapeDtypeStruct((N_TOK, NUM_EXPERTS), jnp.float32),
            jax.ShapeDtypeStruct((N_TOK, 1), jnp.int32),
            jax.ShapeDtypeStruct((N_TOK, 1), jnp.int32),
            jax.ShapeDtypeStruct((N_TOK, 1), jnp.float32),
            jax.ShapeDtypeStruct((N_TOK, 1), jnp.float32),
        ],
    )(x2d, d1, d2, d3, d4, wx, w1, w2, w3, w4, bias_full)


_GRP = 16
_CHUNK = 128
_N_CHUNK = _SLOT_PER_W // _CHUNK
_POS_GRPS = _SLOT_PER_W // _GRP


def _sc_count_body(ex_hbm, cnt_hbm, ex_v, cnt_stage):
    c = lax.axis_index("c")
    s = lax.axis_index("s")
    w = 2 * s + c
    lane = lax.iota(jnp.int32, 16)
    pltpu.sync_copy(ex_hbm.at[pl.ds(512 * w, 512)], ex_v.at[pl.ds(0, 512)])
    zero16 = jnp.zeros((16,), jnp.int32)

    def body(g, a):
        v = ex_v[pl.ds(g * 16, 16)]
        for e in range(NUM_EXPERTS):
            p = jnp.sum((v == e).astype(jnp.int32))
            a = a + jnp.where(lane == e, p, 0)
        return a

    cnt = lax.fori_loop(0, 512 // 16, body, zero16)
    cnt_stage[0, :] = cnt
    pltpu.sync_copy(cnt_stage, cnt_hbm.at[pl.ds(w, 1)])


def _run_sc_count(ex_flat):
    mesh = plsc.VectorSubcoreMesh(core_axis_name="c", subcore_axis_name="s")
    f = pl.kernel(
        _sc_count_body,
        out_type=jax.ShapeDtypeStruct((_NW, 16), jnp.int32),
        mesh=mesh,
        scratch_types=[
            pltpu.VMEM((1024,), jnp.int32),
            pltpu.VMEM((1, 16), jnp.int32),
        ],
        compiler_params=pltpu.CompilerParams(needs_layout_passes=False),
    )
    return f(ex_flat)


def _sc_route_body(ex_hbm, gw_hbm, x_hbm, cnt_hbm,
                   pos_hbm, gwp_hbm, be_hbm, px_hbm,
                   ex_v, all_v, pos_v, gw_v, be_v, xbuf, xbuf2,
                   semg, seml0, seml1, sems0, sems1):
    c = lax.axis_index("c")
    s = lax.axis_index("s")
    w = 2 * s + c
    lane = lax.iota(jnp.int32, 16)
    zero16 = jnp.zeros((16,), jnp.int32)

    pltpu.sync_copy(cnt_hbm, all_v)
    tot = zero16
    pre = zero16
    for r in range(_NW):
        row = all_v[r, :]
        tot = tot + row
        pre = pre + row * (r < w).astype(jnp.int32)
    padded = ((tot + (BLK - 1)) // BLK) * BLK
    base_excl = plsc.cumsum(padded) - padded
    start = base_excl + pre

    pltpu.sync_copy(ex_hbm.at[pl.ds(512 * w, 512)], ex_v.at[pl.ds(0, 512)])

    def pos_grp(g, start_vec):
        v = ex_v[pl.ds(g * 16, 16)]
        posv = jnp.zeros((16,), jnp.int32)
        upd = start_vec
        for e in range(NUM_EXPERTS):
            mi = (v == e).astype(jnp.int32)
            csum = plsc.cumsum(mi)
            start_e = jnp.sum(jnp.where(lane == e, start_vec, 0))
            posv = posv + mi * (start_e + csum - 1)
            cnt_e = jnp.sum(mi)
            upd = upd + jnp.where(lane == e, cnt_e, 0)
        pos_v[g // 4, pl.ds((g % 4) * 16, 16)] = posv
        return upd

    lax.fori_loop(0, _POS_GRPS, pos_grp, start)

    pltpu.sync_copy(pos_v, pos_hbm.at[pl.ds(8 * w, 8)])
    pltpu.sync_copy(gw_hbm.at[pl.ds(8 * w, 8)], gw_v)

    for r in range(8):
        pltpu.async_copy(gw_v.at[r], gwp_hbm.at[pos_v.at[r]], semg)

    @pl.when(w == 0)
    def _():
        base_blk = base_excl // BLK
        sb = [jnp.sum(jnp.where(lane == e, base_blk, 0))
              for e in range(NUM_EXPERTS)]
        for j in range(128 // 16):
            blkid = lane + 16 * j
            bev = jnp.zeros((16,), jnp.int32)
            for e in range(NUM_EXPERTS):
                bev = bev + (blkid >= sb[e]).astype(jnp.int32)
            be_v[pl.ds(16 * j, 16)] = bev - 1
        pltpu.sync_copy(be_v, be_hbm)

    tok_base = 512 * (w % 16)
    xb_sets = ((xbuf, seml0, sems0), (xbuf2, seml1, sems1))
    for ch in range(8):
        xb, sl, ss = xb_sets[ch % 2]
        if ch >= 2:
            pltpu.make_async_copy(xb, px_hbm.at[pos_v.at[ch - 2]], ss).wait()
        pltpu.async_copy(x_hbm.at[pl.ds(tok_base + 64 * ch, 64)], xb, sl)
        pltpu.make_async_copy(x_hbm.at[pl.ds(tok_base + 64 * ch, 64)], xb,
                              sl).wait()
        pltpu.async_copy(xb, px_hbm.at[pos_v.at[ch]], ss)
    for ch in (6, 7):
        xb, _, ss = xb_sets[ch % 2]
        pltpu.make_async_copy(xb, px_hbm.at[pos_v.at[ch]], ss).wait()

    for r in range(8):
        pltpu.make_async_copy(gw_v.at[r], gwp_hbm.at[pos_v.at[r]], semg).wait()


def _run_sc_route(ex_flat, gw_flat, x2d, cnt):
    mesh = plsc.VectorSubcoreMesh(core_axis_name="c", subcore_axis_name="s")
    f = pl.kernel(
        _sc_route_body,
        out_type=[
            jax.ShapeDtypeStruct((N_SLOT // 64, 64), jnp.int32),
            jax.ShapeDtypeStruct((PADN,), jnp.float32),
            jax.ShapeDtypeStruct((128,), jnp.int32),
            jax.ShapeDtypeStruct((PADN, N_EMBD), jnp.float32),
        ],
        mesh=mesh,
        scratch_types=[
            pltpu.VMEM((1024,), jnp.int32),
            pltpu.VMEM((_NW, 16), jnp.int32),
            pltpu.VMEM((8, 64), jnp.int32),
            pltpu.VMEM((8, 64), jnp.float32),
            pltpu.VMEM((128,), jnp.int32),
            pltpu.VMEM((64, N_EMBD), jnp.float32),
            pltpu.VMEM((64, N_EMBD), jnp.float32),
            pltpu.SemaphoreType.DMA,
            pltpu.SemaphoreType.DMA,
            pltpu.SemaphoreType.DMA,
            pltpu.SemaphoreType.DMA,
            pltpu.SemaphoreType.DMA,
        ],
        compiler_params=pltpu.CompilerParams(needs_layout_passes=False),
    )
    return f(ex_flat, gw_flat.reshape(N_SLOT // 64, 64), x2d, cnt)


def _gffn_body(be_ref, x_ref, gw_ref, fcw_ref, fcb_ref, pjw_ref, pjb_ref,
               out_ref):
    xb = x_ref[...].astype(jnp.bfloat16)
    hid = jnp.dot(xb, fcw_ref[0], preferred_element_type=jnp.float32)
    hid = _gelu_tanh(hid + fcb_ref[0])
    out = jnp.dot(hid.astype(jnp.bfloat16), pjw_ref[0],
                  preferred_element_type=jnp.float32)
    out_ref[...] = (out + pjb_ref[0]) * gw_ref[...]


def _run_grouped_ffn(be, perm_x, gw_perm, fc_w, fc_b, proj_w, proj_b):
    grid_spec = pltpu.PrefetchScalarGridSpec(
        num_scalar_prefetch=1,
        grid=(NBLK,),
        in_specs=[
            pl.BlockSpec((BLK, N_EMBD), lambda g, be: (g, 0)),
            pl.BlockSpec((BLK, 1), lambda g, be: (g, 0)),
            pl.BlockSpec((1, N_EMBD, D_FF), lambda g, be: (be[g], 0, 0)),
            pl.BlockSpec((1, 1, D_FF), lambda g, be: (be[g], 0, 0)),
            pl.BlockSpec((1, D_FF, N_EMBD), lambda g, be: (be[g], 0, 0)),
            pl.BlockSpec((1, 1, N_EMBD), lambda g, be: (be[g], 0, 0)),
        ],
        out_specs=pl.BlockSpec((BLK, N_EMBD), lambda g, be: (g, 0)),
    )
    return pl.pallas_call(
        _gffn_body,
        grid_spec=grid_spec,
        out_shape=jax.ShapeDtypeStruct((PADN, N_EMBD), jnp.float32),
    )(be, perm_x, gw_perm.reshape(PADN, 1),
      fc_w.astype(jnp.bfloat16),
      fc_b.reshape(NUM_EXPERTS, 1, D_FF),
      proj_w.astype(jnp.bfloat16),
      proj_b.reshape(NUM_EXPERTS, 1, N_EMBD))


_CTOK = 16


_N_CCH = 256 // _CTOK


def _sc_combine_body(yw_hbm, pos_hbm, out_hbm,
                     idx_v, g0a, g1a, g0b, g1b, sa, sb,
                     semga, semgb, semsa, semsb):
    c = lax.axis_index("c")
    s = lax.axis_index("s")
    w = 2 * s + c
    tok_base = _TOK_PER_W // 2 * w

    pairs = ((g0a, g1a, sa, semga, semsa), (g0b, g1b, sb, semgb, semsb))

    pltpu.sync_copy(pos_hbm.at[pl.ds(4 * w, 4)], idx_v.at[pl.ds(0, 4)])
    pltpu.sync_copy(pos_hbm.at[pl.ds(128 + 4 * w, 4)], idx_v.at[pl.ds(4, 4)])

    def idx0(ch):
        return idx_v.at[ch // 4, pl.ds((ch % 4) * 16, 16)]

    def idx1(ch):
        return idx_v.at[4 + ch // 4, pl.ds((ch % 4) * 16, 16)]

    def start_gather(ch):
        g0, g1, _, sg, _ = pairs[ch % 2]
        pltpu.async_copy(yw_hbm.at[idx0(ch)], g0, sg)
        pltpu.async_copy(yw_hbm.at[idx1(ch)], g1, sg)

    start_gather(0)
    start_gather(1)
    for ch in range(_N_CCH):
        g0, g1, st, sg, ss = pairs[ch % 2]
        pltpu.make_async_copy(yw_hbm.at[idx0(ch)], g0, sg).wait()
        pltpu.make_async_copy(yw_hbm.at[idx1(ch)], g1, sg).wait()
        if ch >= 2:
            tp = tok_base + _CTOK * (ch - 2)
            pltpu.make_async_copy(st, out_hbm.at[pl.ds(tp, _CTOK)], ss).wait()

        def add_row(r, _):
            for cc in range(N_EMBD // 16):
                st[r, pl.ds(cc * 16, 16)] = (g0[r, pl.ds(cc * 16, 16)]
                                             + g1[r, pl.ds(cc * 16, 16)])
            return 0

        lax.fori_loop(0, _CTOK, add_row, 0)
        t0 = tok_base + _CTOK * ch
        pltpu.async_copy(st, out_hbm.at[pl.ds(t0, _CTOK)], ss)
        if ch + 2 < _N_CCH:
            start_gather(ch + 2)
    for ch in (_N_CCH - 2, _N_CCH - 1):
        _, _, st, _, ss = pairs[ch % 2]
        t0 = tok_base + _CTOK * ch
        pltpu.make_async_copy(st, out_hbm.at[pl.ds(t0, _CTOK)], ss).wait()


def _run_sc_combine(yw, pos):
    mesh = plsc.VectorSubcoreMesh(core_axis_name="c", subcore_axis_name="s")
    buf = pltpu.VMEM((_CTOK, N_EMBD), jnp.float32)
    f = pl.kernel(
        _sc_combine_body,
        out_type=jax.ShapeDtypeStruct((N_TOK, N_EMBD), jnp.float32),
        mesh=mesh,
        scratch_types=[
            pltpu.VMEM((8, 64), jnp.int32),
            buf, buf, buf, buf, buf, buf,
            pltpu.SemaphoreType.DMA,
            pltpu.SemaphoreType.DMA,
            pltpu.SemaphoreType.DMA,
            pltpu.SemaphoreType.DMA,
        ],
        compiler_params=pltpu.CompilerParams(needs_layout_passes=False),
    )
    return f(yw, pos)


def kernel(x, city, delta_t_info, delta_dis_info, delta_rg_info,
           delta_entropy_info, city_embeddings, router_w, router_b,
           fc_w, fc_b, proj_w, proj_b):
    b, t, d = x.shape
    n = b * t
    x2d = x.reshape(n, d)
    d1 = delta_t_info.reshape(n, -1)
    d2 = delta_dis_info.reshape(n, -1)
    d3 = delta_rg_info.reshape(n, -1)
    d4 = delta_entropy_info.reshape(n, -1)

    ce = city_embeddings[city]
    w_ce = jax.lax.dynamic_slice_in_dim(router_w, N_EMBD, CITY_DIM, 0)
    bias_full = (jnp.dot(ce, w_ce) + router_b).reshape(1, NUM_EXPERTS)

    gate1, i1, i2, w1, w2 = _run_router(x2d, d1, d2, d3, d4, router_w,
                                        bias_full)
    ex_flat = jnp.concatenate([i1, i2], axis=0).reshape(-1)
    gw_flat = jnp.concatenate([w1, w2], axis=0).reshape(-1)

    cnt = _run_sc_count(ex_flat)
    pos, gw_perm, be_pad, perm_x = _run_sc_route(ex_flat, gw_flat, x2d, cnt)
    yw = _run_grouped_ffn(be_pad[:NBLK], perm_x, gw_perm,
                          fc_w, fc_b, proj_w, proj_b)
    out2d = _run_sc_combine(yw, pos)
    return out2d.reshape(b, t, d), gate1.reshape(b, t, NUM_EXPERTS)


# speedup vs baseline: 2.3319x; 1.1130x over previous
import functools

import jax
import jax.numpy as jnp
from jax import lax
from jax.experimental import pallas as pl
from jax.experimental.pallas import tpu as pltpu
from jax.experimental.pallas import tpu_sc as plsc

B, T, N_EMBD = 4, 2048, 768
NUM_EXPERTS, TOP_K = 8, 2
CITY_DIM = 32
D_FF = 4 * N_EMBD
N_TOK = B * T
N_SLOT = N_TOK * TOP_K

BLK = 256
NBLK = 72
PADN = NBLK * BLK

_NC, _NS = 2, 16
_NW = _NC * _NS
_SLOT_PER_W = N_SLOT // _NW
_TOK_PER_W = _SLOT_PER_W

_SQRT_2_OVER_PI = 0.7978845608028654


def _gelu_tanh(x):
    return 0.5 * x * (1.0 + jnp.tanh(_SQRT_2_OVER_PI * (x + 0.044715 * x * x * x)))


def _router_body(x_ref, d1_ref, d2_ref, d3_ref, d4_ref,
                 wx_ref, w1_ref, w2_ref, w3_ref, w4_ref, bias_ref,
                 gate1_ref, i1_ref, i2_ref, w1o_ref, w2o_ref):
    logits = jnp.dot(x_ref[...], wx_ref[...], preferred_element_type=jnp.float32)
    logits += jnp.dot(d1_ref[...], w1_ref[...], preferred_element_type=jnp.float32)
    logits += jnp.dot(d2_ref[...], w2_ref[...], preferred_element_type=jnp.float32)
    logits += jnp.dot(d3_ref[...], w3_ref[...], preferred_element_type=jnp.float32)
    logits += jnp.dot(d4_ref[...], w4_ref[...], preferred_element_type=jnp.float32)
    logits += bias_ref[...]

    m1 = jnp.max(logits, axis=-1, keepdims=True)
    e = jnp.exp(logits - m1)
    gate1_ref[...] = e / jnp.sum(e, axis=-1, keepdims=True)

    iota = jax.lax.broadcasted_iota(jnp.int32, logits.shape, 1)
    i1 = jnp.min(jnp.where(logits == m1, iota, NUM_EXPERTS), axis=-1,
                 keepdims=True)
    masked = jnp.where(iota == i1, -jnp.inf, logits)
    m2 = jnp.max(masked, axis=-1, keepdims=True)
    i2 = jnp.min(jnp.where(masked == m2, iota, NUM_EXPERTS), axis=-1,
                 keepdims=True)
    w_top1 = 1.0 / (1.0 + jnp.exp(m2 - m1))
    i1_ref[...] = i1
    i2_ref[...] = i2
    w1o_ref[...] = jnp.broadcast_to(w_top1, w1o_ref.shape)
    w2o_ref[...] = jnp.broadcast_to(1.0 - w_top1, w2o_ref.shape)


def _run_router(x2d, d1, d2, d3, d4, router_w, bias_full):
    blk = 1024
    grid = (N_TOK // blk,)
    wx = router_w[:N_EMBD]
    o = N_EMBD + CITY_DIM
    w1 = router_w[o:o + 192]
    w2 = router_w[o + 192:o + 384]
    w3 = router_w[o + 384:o + 480]
    w4 = router_w[o + 480:o + 576]

    def tok_block(i):
        return (i, 0)

    def full(i):
        return (0, 0)

    col = pl.BlockSpec((blk, 1), tok_block)
    wcol = pl.BlockSpec((blk, 16), tok_block)
    return pl.pallas_call(
        _router_body,
        grid=grid,
        in_specs=[
            pl.BlockSpec((blk, N_EMBD), tok_block),
            pl.BlockSpec((blk, 192), tok_block),
            pl.BlockSpec((blk, 192), tok_block),
            pl.BlockSpec((blk, 96), tok_block),
            pl.BlockSpec((blk, 96), tok_block),
            pl.BlockSpec((N_EMBD, NUM_EXPERTS), full),
            pl.BlockSpec((192, NUM_EXPERTS), full),
            pl.BlockSpec((192, NUM_EXPERTS), full),
            pl.BlockSpec((96, NUM_EXPERTS), full),
            pl.BlockSpec((96, NUM_EXPERTS), full),
            pl.BlockSpec((1, NUM_EXPERTS), full),
        ],
        out_specs=[
            pl.BlockSpec((blk, NUM_EXPERTS), tok_block),
            col, col, wcol, wcol,
        ],
        out_shape=[
            jax.ShapeDtypeStruct((N_TOK, NUM_EXPERTS), jnp.float32),
            jax.ShapeDtypeStruct((N_TOK, 1), jnp.int32),
            jax.ShapeDtypeStruct((N_TOK, 1), jnp.int32),
            jax.ShapeDtypeStruct((N_TOK, 16), jnp.float32),
            jax.ShapeDtypeStruct((N_TOK, 16), jnp.float32),
        ],
    )(x2d, d1, d2, d3, d4, wx, w1, w2, w3, w4, bias_full)


_GRP = 16
_CHUNK = 128
_N_CHUNK = _SLOT_PER_W // _CHUNK
_POS_GRPS = _SLOT_PER_W // _GRP


def _sc_count_body(ex0_hbm, ex1_hbm, cnt_hbm, ex_v, cnt_stage):
    c = lax.axis_index("c")
    s = lax.axis_index("s")
    w = 2 * s + c
    lane = lax.iota(jnp.int32, 16)

    @pl.when(w < 16)
    def _():
        pltpu.sync_copy(ex0_hbm.at[pl.ds(512 * w, 512)],
                        ex_v.at[pl.ds(0, 512)])

    @pl.when(w >= 16)
    def _():
        pltpu.sync_copy(ex1_hbm.at[pl.ds(512 * (w - 16), 512)],
                        ex_v.at[pl.ds(0, 512)])
    zero16 = jnp.zeros((16,), jnp.int32)

    def body(g, a):
        v = ex_v[pl.ds(g * 16, 16)]
        for e in range(NUM_EXPERTS):
            p = jnp.sum((v == e).astype(jnp.int32))
            a = a + jnp.where(lane == e, p, 0)
        return a

    cnt = lax.fori_loop(0, 512 // 16, body, zero16)
    cnt_stage[0, :] = cnt
    pltpu.sync_copy(cnt_stage, cnt_hbm.at[pl.ds(w, 1)])


def _run_sc_count(ex0, ex1):
    mesh = plsc.VectorSubcoreMesh(core_axis_name="c", subcore_axis_name="s")
    f = pl.kernel(
        _sc_count_body,
        out_type=jax.ShapeDtypeStruct((_NW, 16), jnp.int32),
        mesh=mesh,
        scratch_types=[
            pltpu.VMEM((1024,), jnp.int32),
            pltpu.VMEM((1, 16), jnp.int32),
        ],
        compiler_params=pltpu.CompilerParams(needs_layout_passes=False),
    )
    return f(ex0, ex1)


def _sc_route_body(ex0_hbm, ex1_hbm, x_hbm, cnt_hbm,
                   pos_hbm, be_hbm, px_hbm,
                   ex_v, all_v, pos_v, be_v, xbuf, xbuf2,
                   seml0, seml1, sems0, sems1):
    c = lax.axis_index("c")
    s = lax.axis_index("s")
    w = 2 * s + c
    lane = lax.iota(jnp.int32, 16)
    zero16 = jnp.zeros((16,), jnp.int32)

    pltpu.sync_copy(cnt_hbm, all_v)
    tot = zero16
    pre = zero16
    for r in range(_NW):
        row = all_v[r, :]
        tot = tot + row
        pre = pre + row * (r < w).astype(jnp.int32)
    padded = ((tot + (BLK - 1)) // BLK) * BLK
    base_excl = plsc.cumsum(padded) - padded
    start = base_excl + pre

    @pl.when(w < 16)
    def _():
        pltpu.sync_copy(ex0_hbm.at[pl.ds(512 * w, 512)],
                        ex_v.at[pl.ds(0, 512)])

    @pl.when(w >= 16)
    def _():
        pltpu.sync_copy(ex1_hbm.at[pl.ds(512 * (w - 16), 512)],
                        ex_v.at[pl.ds(0, 512)])

    def pos_grp(g, start_vec):
        v = ex_v[pl.ds(g * 16, 16)]
        posv = jnp.zeros((16,), jnp.int32)
        upd = start_vec
        for e in range(NUM_EXPERTS):
            mi = (v == e).astype(jnp.int32)
            csum = plsc.cumsum(mi)
            start_e = jnp.sum(jnp.where(lane == e, start_vec, 0))
            posv = posv + mi * (start_e + csum - 1)
            cnt_e = jnp.sum(mi)
            upd = upd + jnp.where(lane == e, cnt_e, 0)
        pos_v[g // 4, pl.ds((g % 4) * 16, 16)] = posv
        return upd

    lax.fori_loop(0, _POS_GRPS, pos_grp, start)

    pltpu.sync_copy(pos_v, pos_hbm.at[pl.ds(8 * w, 8)])

    @pl.when(w == 0)
    def _():
        base_blk = base_excl // BLK
        sb = [jnp.sum(jnp.where(lane == e, base_blk, 0))
              for e in range(NUM_EXPERTS)]
        for j in range(128 // 16):
            blkid = lane + 16 * j
            bev = jnp.zeros((16,), jnp.int32)
            for e in range(NUM_EXPERTS):
                bev = bev + (blkid >= sb[e]).astype(jnp.int32)
            be_v[pl.ds(16 * j, 16)] = bev - 1
        pltpu.sync_copy(be_v, be_hbm)

    tok_base = 512 * (w % 16)
    xb_sets = ((xbuf, seml0, sems0), (xbuf2, seml1, sems1))
    for ch in range(8):
        xb, sl, ss = xb_sets[ch % 2]
        if ch >= 2:
            pltpu.make_async_copy(xb, px_hbm.at[pos_v.at[ch - 2]], ss).wait()
        pltpu.async_copy(x_hbm.at[pl.ds(tok_base + 64 * ch, 64)], xb, sl)
        pltpu.make_async_copy(x_hbm.at[pl.ds(tok_base + 64 * ch, 64)], xb,
                              sl).wait()
        pltpu.async_copy(xb, px_hbm.at[pos_v.at[ch]], ss)
    for ch in (6, 7):
        xb, _, ss = xb_sets[ch % 2]
        pltpu.make_async_copy(xb, px_hbm.at[pos_v.at[ch]], ss).wait()


def _run_sc_route(ex0, ex1, x2d, cnt):
    mesh = plsc.VectorSubcoreMesh(core_axis_name="c", subcore_axis_name="s")
    f = pl.kernel(
        _sc_route_body,
        out_type=[
            jax.ShapeDtypeStruct((N_SLOT // 64, 64), jnp.int32),
            jax.ShapeDtypeStruct((128,), jnp.int32),
            jax.ShapeDtypeStruct((PADN, N_EMBD), jnp.float32),
        ],
        mesh=mesh,
        scratch_types=[
            pltpu.VMEM((1024,), jnp.int32),
            pltpu.VMEM((_NW, 16), jnp.int32),
            pltpu.VMEM((8, 64), jnp.int32),
            pltpu.VMEM((128,), jnp.int32),
            pltpu.VMEM((64, N_EMBD), jnp.float32),
            pltpu.VMEM((64, N_EMBD), jnp.float32),
            pltpu.SemaphoreType.DMA,
            pltpu.SemaphoreType.DMA,
            pltpu.SemaphoreType.DMA,
            pltpu.SemaphoreType.DMA,
        ],
        compiler_params=pltpu.CompilerParams(needs_layout_passes=False),
    )
    return f(ex0, ex1, x2d, cnt)


def _gffn_body(be_ref, x_ref, fcw_ref, fcb_ref, pjw_ref, pjb_ref,
               out_ref):
    xb = x_ref[...].astype(jnp.bfloat16)
    hid = jnp.dot(xb, fcw_ref[0], preferred_element_type=jnp.float32)
    hid = _gelu_tanh(hid + fcb_ref[0])
    out = jnp.dot(hid.astype(jnp.bfloat16), pjw_ref[0],
                  preferred_element_type=jnp.float32)
    out_ref[...] = out + pjb_ref[0]


def _run_grouped_ffn(be, perm_x, fc_w, fc_b, proj_w, proj_b):
    grid_spec = pltpu.PrefetchScalarGridSpec(
        num_scalar_prefetch=1,
        grid=(NBLK,),
        in_specs=[
            pl.BlockSpec((BLK, N_EMBD), lambda g, be: (g, 0)),
            pl.BlockSpec((1, N_EMBD, D_FF), lambda g, be: (be[g], 0, 0)),
            pl.BlockSpec((1, 1, D_FF), lambda g, be: (be[g], 0, 0)),
            pl.BlockSpec((1, D_FF, N_EMBD), lambda g, be: (be[g], 0, 0)),
            pl.BlockSpec((1, 1, N_EMBD), lambda g, be: (be[g], 0, 0)),
        ],
        out_specs=pl.BlockSpec((BLK, N_EMBD), lambda g, be: (g, 0)),
    )
    return pl.pallas_call(
        _gffn_body,
        grid_spec=grid_spec,
        out_shape=jax.ShapeDtypeStruct((PADN, N_EMBD), jnp.float32),
    )(be, perm_x,
      fc_w.astype(jnp.bfloat16),
      fc_b.reshape(NUM_EXPERTS, 1, D_FF),
      proj_w.astype(jnp.bfloat16),
      proj_b.reshape(NUM_EXPERTS, 1, N_EMBD))


_CTOK = 16


_N_CCH = 256 // _CTOK


def _sc_combine_body(yw_hbm, pos_hbm, gw0_hbm, gw1_hbm, out_hbm,
                     idx_v, gw0_v, gw1_v, g0a, g1a, g0b, g1b, st,
                     semga, semgb, sems):
    c = lax.axis_index("c")
    s = lax.axis_index("s")
    w = 2 * s + c
    tok_base = _TOK_PER_W // 2 * w

    pairs = ((g0a, g1a, semga), (g0b, g1b, semgb))

    pltpu.sync_copy(pos_hbm.at[pl.ds(4 * w, 4)], idx_v.at[pl.ds(0, 4)])
    pltpu.sync_copy(pos_hbm.at[pl.ds(128 + 4 * w, 4)], idx_v.at[pl.ds(4, 4)])
    pltpu.sync_copy(gw0_hbm.at[pl.ds(tok_base, 256)], gw0_v)
    pltpu.sync_copy(gw1_hbm.at[pl.ds(tok_base, 256)], gw1_v)

    def idx0(ch):
        return idx_v.at[ch // 4, pl.ds((ch % 4) * 16, 16)]

    def idx1(ch):
        return idx_v.at[4 + ch // 4, pl.ds((ch % 4) * 16, 16)]

    def start_gather(ch):
        g0, g1, sg = pairs[ch % 2]
        pltpu.async_copy(yw_hbm.at[idx0(ch)], g0, sg)
        pltpu.async_copy(yw_hbm.at[idx1(ch)], g1, sg)

    start_gather(0)
    start_gather(1)
    for ch in range(_N_CCH):
        g0, g1, sg = pairs[ch % 2]
        pltpu.make_async_copy(yw_hbm.at[idx0(ch)], g0, sg).wait()
        pltpu.make_async_copy(yw_hbm.at[idx1(ch)], g1, sg).wait()
        if ch >= 1:
            tp = tok_base + _CTOK * (ch - 1)
            pltpu.make_async_copy(st, out_hbm.at[pl.ds(tp, _CTOK)],
                                  sems).wait()

        def add_row(r, _):
            gv0 = gw0_v[_CTOK * ch + r, :]
            gv1 = gw1_v[_CTOK * ch + r, :]
            for cc in range(N_EMBD // 16):
                st[r, pl.ds(cc * 16, 16)] = (g0[r, pl.ds(cc * 16, 16)] * gv0
                                             + g1[r, pl.ds(cc * 16, 16)] * gv1)
            return 0

        lax.fori_loop(0, _CTOK, add_row, 0)
        t0 = tok_base + _CTOK * ch
        pltpu.async_copy(st, out_hbm.at[pl.ds(t0, _CTOK)], sems)
        if ch + 2 < _N_CCH:
            start_gather(ch + 2)
    t0 = tok_base + _CTOK * (_N_CCH - 1)
    pltpu.make_async_copy(st, out_hbm.at[pl.ds(t0, _CTOK)], sems).wait()


def _run_sc_combine(yw, pos, gw0, gw1):
    mesh = plsc.VectorSubcoreMesh(core_axis_name="c", subcore_axis_name="s")
    buf = pltpu.VMEM((_CTOK, N_EMBD), jnp.float32)
    f = pl.kernel(
        _sc_combine_body,
        out_type=jax.ShapeDtypeStruct((N_TOK, N_EMBD), jnp.float32),
        mesh=mesh,
        scratch_types=[
            pltpu.VMEM((8, 64), jnp.int32),
            pltpu.VMEM((256, 16), jnp.float32),
            pltpu.VMEM((256, 16), jnp.float32),
            buf, buf, buf, buf, buf,
            pltpu.SemaphoreType.DMA,
            pltpu.SemaphoreType.DMA,
            pltpu.SemaphoreType.DMA,
        ],
        compiler_params=pltpu.CompilerParams(needs_layout_passes=False),
    )
    return f(yw, pos, gw0, gw1)


def kernel(x, city, delta_t_info, delta_dis_info, delta_rg_info,
           delta_entropy_info, city_embeddings, router_w, router_b,
           fc_w, fc_b, proj_w, proj_b):
    b, t, d = x.shape
    n = b * t
    x2d = x.reshape(n, d)
    d1 = delta_t_info.reshape(n, -1)
    d2 = delta_dis_info.reshape(n, -1)
    d3 = delta_rg_info.reshape(n, -1)
    d4 = delta_entropy_info.reshape(n, -1)

    ce = city_embeddings[city]
    w_ce = jax.lax.dynamic_slice_in_dim(router_w, N_EMBD, CITY_DIM, 0)
    bias_full = (jnp.dot(ce, w_ce) + router_b).reshape(1, NUM_EXPERTS)

    gate1, i1, i2, w1, w2 = _run_router(x2d, d1, d2, d3, d4, router_w,
                                        bias_full)
    ex0 = i1.reshape(-1)
    ex1 = i2.reshape(-1)

    cnt = _run_sc_count(ex0, ex1)
    pos, be_pad, perm_x = _run_sc_route(ex0, ex1, x2d, cnt)
    yw = _run_grouped_ffn(be_pad[:NBLK], perm_x, fc_w, fc_b, proj_w, proj_b)
    out2d = _run_sc_combine(yw, pos, w1, w2)
    return out2d.reshape(b, t, d), gate1.reshape(b, t, NUM_EXPERTS)
